# Initial kernel scaffold; baseline (speedup 1.0000x reference)
import jax
import jax.numpy as jnp
from jax.experimental import pallas as pl


def kernel(x, pos, edge_attr, edge_index, batch_idx, task_id, Wemb, bemb, We1, be1, We2, be2, Wx1, bx1, Wx2, bx2, Wh1, bh1, Wh2, bh2, Wha, bha, Whb, bhb):
    raise NotImplementedError("write your pallas kernel here")


# speedup vs baseline: 1.8769x; 1.8769x over previous
import functools

import jax
import jax.numpy as jnp
from jax import lax
from jax.experimental import pallas as pl
from jax.experimental.pallas import tpu as pltpu
from jax.experimental.pallas import tpu_sc as plsc

F32 = jnp.float32
TW = 256
PW = 144
NBLK = 1000
EBLK = 640


def _silu(v):
    return v / (1.0 + jnp.exp(-v))


def _dot(a, b):
    return jnp.dot(a, b, preferred_element_type=F32)


def _pre_body(x_ref, p8_ref, wemb_ref, bemb_ref, wd_ref, ws_ref,
              h_ref, td_ref, ts_ref):
    h = _dot(x_ref[...], wemb_ref[...]) + bemb_ref[...]
    h_ref[...] = h
    p8 = p8_ref[...]
    z = jnp.zeros((h.shape[0], TW - 136), F32)
    td_ref[...] = jnp.concatenate([_dot(h, wd_ref[...]), p8, z], axis=1)
    ts_ref[...] = jnp.concatenate([_dot(h, ws_ref[...]), p8, z], axis=1)


_SEL48 = None


def _edge_body(d_ref, s_ref, ea_ref, wex_ref, be1_ref, we2_ref, be2_ref,
               wx1_ref, bx1_ref, wx2_ref, bx2_ref, m_ref, aux_ref):
    d = d_ref[...]
    s = s_ref[...]
    u = d[:, :128] + s[:, :128]
    relp = d[:, 128:136] - s[:, 128:136]
    d2 = jnp.sum(relp * relp, axis=1, keepdims=True)
    extra = jnp.concatenate([d2, ea_ref[...]], axis=1)
    m1 = _silu(u + _dot(extra, wex_ref[...]) + be1_ref[...])
    m = _silu(_dot(m1, we2_ref[...]) + be2_ref[...])
    t1 = _silu(_dot(m, wx1_ref[...]) + bx1_ref[...])
    xw = jnp.sum(t1 * wx2_ref[...], axis=1, keepdims=True) + bx2_ref[...]
    m_ref[...] = m
    rx = relp * xw
    sel = jnp.concatenate(
        [jnp.eye(3, 8, dtype=F32), jnp.zeros((1, 8), F32)], axis=0)
    aux = lax.dot_general(sel, rx, (((1,), (1,)), ((), ())),
                          preferred_element_type=F32)
    aux_ref[...] = aux + jnp.concatenate(
        [jnp.zeros((3, aux.shape[1]), F32), jnp.ones((1, aux.shape[1]), F32)], axis=0)


def _node_body(a_ref, x_ref, h_ref, p8_ref, wh1a_ref,
               wh1b_ref, bh1_ref, wh2_ref, bh2_ref, wd_ref, ws_ref,
               hn_ref, pn_ref, td_ref, ts_ref):
    aggm = a_ref[0]
    small = x_ref[0]
    deg = small[:, 3:4]
    n = small.shape[0]
    aggx = jnp.concatenate([small[:, :3], jnp.zeros((n, 5), F32)], axis=1)
    p_new = p8_ref[...] + aggx / (deg + 1.0)
    h = h_ref[...]
    hu = _silu(_dot(h, wh1a_ref[...]) + _dot(aggm, wh1b_ref[...]) + bh1_ref[...])
    h_new = h + _dot(hu, wh2_ref[...]) + bh2_ref[...]
    hn_ref[...] = h_new
    pn_ref[...] = p_new
    if td_ref is not None:
        z = jnp.zeros((h.shape[0], TW - 136), F32)
        td_ref[...] = jnp.concatenate([_dot(h_new, wd_ref[...]), p_new, z], axis=1)
        ts_ref[...] = jnp.concatenate([_dot(h_new, ws_ref[...]), p_new, z], axis=1)


def _node_last_body(a_ref, h_ref, wh1a_ref, wh1b_ref, bh1_ref,
                    wh2_ref, bh2_ref, hn_ref):
    aggm = a_ref[0]
    h = h_ref[...]
    hu = _silu(_dot(h, wh1a_ref[...]) + _dot(aggm, wh1b_ref[...]) + bh1_ref[...])
    hn_ref[...] = h + _dot(hu, wh2_ref[...]) + bh2_ref[...]


def _pool_body(h_ref, bf_ref, g_ref):
    i = pl.program_id(0)

    @pl.when(i == 0)
    def _():
        g_ref[...] = jnp.zeros_like(g_ref)

    n = h_ref.shape[0]
    bf = bf_ref[0]
    rows = lax.broadcasted_iota(jnp.int32, (128, n), 0).astype(F32)
    onehot = jnp.where(rows == bf, 1.0, 0.0)
    hb = jnp.concatenate([h_ref[...], jnp.ones((n, 16), F32)], axis=1)
    g_ref[...] += _dot(onehot, hb)


def _head_body(g_ref, tid_ref, wha_ref, bha_ref, whb_ref, bhb_ref, out_ref):
    ga = g_ref[...]
    cnt = jnp.maximum(ga[:, 128:129], 1.0)
    g = ga[:, :128] / cnt
    tid = tid_ref[...]
    nt = wha_ref.shape[0]
    logits = jnp.zeros((g.shape[0], 1), F32)
    for t in range(nt):
        hid = _silu(_dot(g, wha_ref[t]) + bha_ref[t][None, :])
        o = jnp.sum(hid * whb_ref[t][None, :], axis=1, keepdims=True) + bhb_ref[t, 0]
        logits = jnp.where(tid == t, o, logits)
    out_ref[...] = logits


def _sc_mesh():
    return plsc.VectorSubcoreMesh(core_axis_name="c", subcore_axis_name="s")


def _make_gather(E):
    nch = E // 128

    @functools.partial(
        pl.kernel,
        out_type=(jax.ShapeDtypeStruct((E, TW), F32),
                  jax.ShapeDtypeStruct((E, TW), F32)),
        mesh=_sc_mesh(),
        scratch_types=[
            pltpu.VMEM((128,), jnp.int32), pltpu.VMEM((128,), jnp.int32),
            pltpu.VMEM((128, TW), F32), pltpu.VMEM((128, TW), F32),
            pltpu.SemaphoreType.DMA, pltpu.SemaphoreType.DMA,
        ],
    )
    def gath(tbl_d, tbl_s, dst2, src2, out_d, out_s, di_v, si_v, db_v, sb_v,
             sem_d, sem_s):
        wid = lax.axis_index("s") * 2 + lax.axis_index("c")

        @pl.loop(wid, nch, step=32)
        def _(ci):
            pltpu.sync_copy(dst2.at[ci], di_v)
            pltpu.sync_copy(src2.at[ci], si_v)
            cp_d = pltpu.async_copy(tbl_d.at[di_v], db_v, sem_d)
            cp_s = pltpu.async_copy(tbl_s.at[si_v], sb_v, sem_s)
            cp_d.wait()
            cp_s.wait()
            pltpu.sync_copy(db_v, out_d.at[pl.ds(ci * 128, 128)])
            pltpu.sync_copy(sb_v, out_s.at[pl.ds(ci * 128, 128)])

    return gath


def _make_scatter(E, N):
    nch = E // 128
    hn = N // 2
    hnp = ((hn + 64 + 127) // 128) * 128
    rpt = hnp // 16

    @functools.partial(
        pl.kernel,
        out_type=(jax.ShapeDtypeStruct((2 * hnp, 128), F32),
                  jax.ShapeDtypeStruct((2 * hnp * 4,), F32)),
        mesh=_sc_mesh(),
        scratch_types=[
            pltpu.VMEM((128,), jnp.int32), pltpu.VMEM((128,), jnp.int32),
            pltpu.VMEM((128,), jnp.int32), pltpu.VMEM((128,), jnp.int32),
            pltpu.VMEM((128, 128), F32), pltpu.VMEM((128,), F32),
            pltpu.VMEM((hnp * 4,), F32),
            pltpu.VMEM_SHARED((hnp, 128), F32),
            pltpu.VMEM_SHARED((hnp * 4,), F32),
        ],
    )
    def scat(m_rows, aux3, dst2, zeros_nw, zeros_x, out_m, out_x,
             di_v, mi_v, xb_i, xi_v, mb_v, xb_v, xd_v, acc_sh, acx_sh):
        c0 = lax.axis_index("c")
        s0 = lax.axis_index("s")
        base = s0 * rpt
        lo = c0 * hn

        pltpu.sync_copy(zeros_nw.at[pl.ds(base, rpt)],
                        acc_sh.at[pl.ds(base, rpt)])

        @pl.when(s0 == 0)
        def _():
            pltpu.sync_copy(zeros_x, xd_v)
            pltpu.sync_copy(xd_v, acx_sh)

        plsc.subcore_barrier()

        @pl.loop(s0, nch, step=16)
        def _(ci):
            pltpu.sync_copy(dst2.at[ci], di_v)
            pltpu.sync_copy(m_rows.at[pl.ds(ci * 128, 128)], mb_v)
            for j in range(8):
                sl = pl.ds(j * 16, 16)
                di = di_v[sl]
                off = di - lo
                ok = (off >= 0) & (off < hn)
                mi_v[sl] = jnp.where(ok, off, hn + (di & 63))
                xb_i[sl] = jnp.where(ok, off * 4, hn * 4 + (di & 255))
            pltpu.sync_copy(mb_v, acc_sh.at[mi_v], add=True)
            for k in range(4):
                for j in range(8):
                    sl = pl.ds(j * 16, 16)
                    xi_v[sl] = xb_i[sl] + k
                pltpu.sync_copy(aux3.at[k, ci], xb_v)
                pltpu.sync_copy(xb_v, acx_sh.at[xi_v], add=True)

        plsc.subcore_barrier()

        pltpu.sync_copy(acc_sh.at[pl.ds(base, rpt)],
                        out_m.at[pl.ds(c0 * hnp + base, rpt)])

        @pl.when(s0 == 1)
        def _():
            pltpu.sync_copy(acx_sh, xd_v)
            pltpu.sync_copy(xd_v, out_x.at[pl.ds(c0 * hnp * 4, hnp * 4)])

    return scat


def kernel(x, pos, edge_attr, edge_index, batch_idx, task_id, Wemb, bemb,
           We1, be1, We2, be2, Wx1, bx1, Wx2, bx2, Wh1, bh1, Wh2, bh2,
           Wha, bha, Whb, bhb):
    N, ND = x.shape
    E, ED = edge_attr.shape
    B = task_id.shape[0]
    H = Wemb.shape[1]
    L = We1.shape[0]

    src2 = edge_index[0].reshape(E // 128, 128)
    dst2 = edge_index[1].reshape(E // 128, 128)
    p8 = jnp.pad(pos, ((0, 0), (0, 8 - pos.shape[1])))
    batch_f = batch_idx.astype(F32).reshape(N // NBLK, 1, NBLK)
    tid2 = task_id.reshape(B, 1)
    hn = N // 2
    hnp = ((hn + 64 + 127) // 128) * 128
    zeros_nw = jnp.zeros((N, 128), F32)
    zeros_x = jnp.zeros((hnp * 4,), F32)

    w1d = We1[:, :H, :]
    w1s = We1[:, H:2 * H, :]
    w1x = We1[:, 2 * H:, :]
    wh1a = Wh1[:, :H, :]
    wh1b = Wh1[:, H:, :]
    wx2r = Wx2.reshape(L, 1, H)
    whbr = Whb.reshape(Whb.shape[0], Whb.shape[1])

    gn = N // NBLK
    ge = E // EBLK

    full = lambda shape: pl.BlockSpec(shape, lambda *_: tuple(0 for _ in shape))
    rowsN = lambda w: pl.BlockSpec((NBLK, w), lambda i: (i, 0))
    rowsE = lambda w: pl.BlockSpec((EBLK, w), lambda i: (i, 0))

    h, tbl_d, tbl_s = pl.pallas_call(
        _pre_body,
        grid=(gn,),
        in_specs=[rowsN(ND), rowsN(8), full((ND, H)), full((1, H)),
                  full((H, H)), full((H, H))],
        out_specs=[rowsN(H), rowsN(TW), rowsN(TW)],
        out_shape=[jax.ShapeDtypeStruct((N, H), F32),
                   jax.ShapeDtypeStruct((N, TW), F32),
                   jax.ShapeDtypeStruct((N, TW), F32)],
    )(x, p8, Wemb, bemb.reshape(1, H), w1d[0], w1s[0])

    gather = _make_gather(E)
    scatter = _make_scatter(E, N)

    p8_cur = p8
    for l in range(L):
        gd, gs = gather(tbl_d, tbl_s, dst2, src2)

        m_rows, aux = pl.pallas_call(
            _edge_body,
            grid=(ge,),
            in_specs=[rowsE(TW), rowsE(TW), rowsE(ED), full((1 + ED, H)),
                      full((1, H)), full((H, H)), full((1, H)),
                      full((H, H)), full((1, H)), full((1, H)), full((1, 1))],
            out_specs=[rowsE(128), pl.BlockSpec((4, EBLK), lambda i: (0, i))],
            out_shape=[jax.ShapeDtypeStruct((E, 128), F32),
                       jax.ShapeDtypeStruct((4, E), F32)],
        )(gd, gs, edge_attr, w1x[l], be1[l].reshape(1, H), We2[l],
          be2[l].reshape(1, H), Wx1[l], bx1[l].reshape(1, H), wx2r[l],
          bx2[l].reshape(1, 1))

        acc_m, acc_x = scatter(m_rows, aux.reshape(4, E // 128, 128), dst2,
                               zeros_nw, zeros_x)
        accm3 = acc_m.reshape(2, hnp, 128)
        accx3 = acc_x.reshape(2, hnp, 4)

        hb = gn // 2
        a_spec = pl.BlockSpec((1, NBLK, 128), lambda i: (i // hb, i % hb, 0))
        x_spec = pl.BlockSpec((1, NBLK, 4), lambda i: (i // hb, i % hb, 0))
        if l < L - 1:
            h, p8_cur, tbl_d, tbl_s = pl.pallas_call(
                _node_body,
                grid=(gn,),
                in_specs=[a_spec, x_spec, rowsN(H), rowsN(8),
                          full((H, H)), full((H, H)), full((1, H)),
                          full((H, H)), full((1, H)),
                          full((H, H)), full((H, H))],
                out_specs=[rowsN(H), rowsN(8), rowsN(TW), rowsN(TW)],
                out_shape=[jax.ShapeDtypeStruct((N, H), F32),
                           jax.ShapeDtypeStruct((N, 8), F32),
                           jax.ShapeDtypeStruct((N, TW), F32),
                           jax.ShapeDtypeStruct((N, TW), F32)],
            )(accm3, accx3, h, p8_cur, wh1a[l], wh1b[l],
              bh1[l].reshape(1, H), Wh2[l], bh2[l].reshape(1, H),
              w1d[l + 1], w1s[l + 1])
        else:
            h = pl.pallas_call(
                _node_last_body,
                grid=(gn,),
                in_specs=[a_spec, rowsN(H),
                          full((H, H)), full((H, H)), full((1, H)),
                          full((H, H)), full((1, H))],
                out_specs=rowsN(H),
                out_shape=jax.ShapeDtypeStruct((N, H), F32),
            )(accm3, h, wh1a[l], wh1b[l], bh1[l].reshape(1, H),
              Wh2[l], bh2[l].reshape(1, H))

    g_aug = pl.pallas_call(
        _pool_body,
        grid=(gn,),
        in_specs=[rowsN(H), pl.BlockSpec((1, 1, NBLK), lambda i: (i, 0, 0))],
        out_specs=pl.BlockSpec((B, PW), lambda i: (0, 0)),
        out_shape=jax.ShapeDtypeStruct((B, PW), F32),
    )(h, batch_f)

    logits = pl.pallas_call(
        _head_body,
        in_specs=[full((B, PW)), full((B, 1)), full(Wha.shape), full(bha.shape),
                  full(whbr.shape), full(bhb.shape)],
        out_specs=full((B, 1)),
        out_shape=jax.ShapeDtypeStruct((B, 1), F32),
    )(g_aug, tid2, Wha, bha, whbr, bhb)

    return logits
